# baseline (speedup 1.0000x reference)
import functools

import jax
import jax.numpy as jnp
from jax import lax
from jax.experimental import pallas as pl
from jax.experimental.pallas import tpu as pltpu
from jax.experimental.pallas import tpu_sc as plsc

_D = 128
_DT = 64
_KT = 16
_UP = 4
_B = 2
_M = 2048
_XP = 16
_G = 256
_TP = 512
_TK = 256
_T = 256
_R = _T * _KT

_NC = 2
_NS = 16
_NW = _NC * _NS
_PC = 2
_MC = _M // _PC
_N = _B * _MC * _KT
_PW = _N // _NW
_CH = 128


def _dot(a, b, ca, cb):
    return lax.dot_general(a, b, (((ca,), (cb,)), ((), ())),
                           preferred_element_type=jnp.float32)


def _preamble_body(fq_ref, fk_ref, xyz_ref, w1q_ref, w1k_ref, b1_ref, w2_ref,
                   b2_ref, wrq_ref, wrk_ref, br_ref, qw_ref, qb_ref, kw_ref,
                   kb_ref, vw_ref, vb_ref, pw1_ref,
                   q_out, tab_out, resi_out):
    fq = fq_ref[0]
    fk = fk_ref[0]
    h = _dot(w1q_ref[...], fq, 1, 0) + _dot(w1k_ref[...], fk, 1, 0)
    h = jax.nn.relu(h + b1_ref[...])
    fv = (_dot(h, w2_ref[...], 0, 1) + _dot(fq, wrq_ref[...], 0, 1)
          + _dot(fk, wrk_ref[...], 0, 1) + b2_ref[...] + br_ref[...])
    resi_out[0] = fv
    q_out[0] = _dot(fq, qw_ref[...], 0, 1) + qb_ref[...]
    kk = _dot(fk, kw_ref[...], 0, 1) + kb_ref[...]
    vv = _dot(fv, vw_ref[...], 1, 1) + vb_ref[...]
    p1 = _dot(xyz_ref[0], pw1_ref[...], 1, 1)
    tab_out[0] = jnp.concatenate(
        [kk, vv, p1, jnp.zeros((_TP, _DT), jnp.float32)], axis=1)


def _preamble(fts_q, fts_k, xyzp, mv_w1, mv_b1, mv_w2, mv_b2, mv_wres,
              mv_bres, q_w, q_b, k_w, k_b, v_w, v_b, pos_w1p):
    grid = (_B, _M // _TP)
    io = lambda b, t: (b, 0, t)
    w = lambda *s: pl.BlockSpec(s, lambda b, t: (0,) * len(s))
    return pl.pallas_call(
        _preamble_body,
        grid=grid,
        compiler_params=pltpu.CompilerParams(
            dimension_semantics=("parallel", "parallel")),
        in_specs=[
            pl.BlockSpec((1, _D, _TP), io),
            pl.BlockSpec((1, _D, _TP), io),
            pl.BlockSpec((1, _TP, _XP), lambda b, t: (b, t, 0)),
            w(_D, _D), w(_D, _D), w(_D, 1), w(_D, _D), w(1, _D),
            w(_D, _D), w(_D, _D), w(1, _D),
            w(_DT, _D), w(1, _DT), w(_DT, _D), w(1, _DT), w(_DT, _D), w(1, _DT),
            w(_DT, _XP),
        ],
        out_specs=[
            pl.BlockSpec((1, _TP, _DT), lambda b, t: (b, t, 0)),
            pl.BlockSpec((1, _TP, _G), lambda b, t: (b, t, 0)),
            pl.BlockSpec((1, _TP, _D), lambda b, t: (b, t, 0)),
        ],
        out_shape=[
            jax.ShapeDtypeStruct((_B, _M, _DT), jnp.float32),
            jax.ShapeDtypeStruct((_B, _M, _G), jnp.float32),
            jax.ShapeDtypeStruct((_B, _M, _D), jnp.float32),
        ],
    )(fts_q, fts_k, xyzp,
      mv_w1[:, :_D], mv_w1[:, _D:], mv_b1[:, None], mv_w2, mv_b2[None, :],
      mv_wres[:, :_D], mv_wres[:, _D:], mv_bres[None, :],
      q_w, q_b[None, :], k_w, k_b[None, :], v_w, v_b[None, :], pos_w1p)


def _knn_body(xt_ref, xa_ref, ids_out):
    b = pl.program_id(0)
    xt = xt_ref[0]
    xa = xa_ref[0]
    sqt = jnp.sum(xt * xt, axis=1, keepdims=True)
    sqa = jnp.sum(xa * xa, axis=0, keepdims=True)
    d = sqt + sqa - 2.0 * _dot(xt, xa, 1, 0)
    iota = lax.broadcasted_iota(jnp.int32, (_TK, _M), 1)
    cols = []
    for _ in range(_KT):
        idx = jnp.argmin(d, axis=1).astype(jnp.int32)[:, None]
        cols.append(idx)
        d = jnp.where(iota == idx, 3.0e38, d)
    ids = jnp.concatenate(cols, axis=1)
    ids_out[0] = ids + b * _M


def _knn(xyzp, xyzpT, c):
    toff = c * (_MC // _TK)
    return pl.pallas_call(
        _knn_body,
        grid=(_B, _MC // _TK),
        compiler_params=pltpu.CompilerParams(
            dimension_semantics=("parallel", "parallel")),
        in_specs=[
            pl.BlockSpec((1, _TK, _XP), lambda b, t: (b, toff + t, 0)),
            pl.BlockSpec((1, _XP, _M), lambda b, t: (b, 0, 0)),
        ],
        out_specs=pl.BlockSpec((1, _TK, _KT), lambda b, t: (b, t, 0)),
        out_shape=jax.ShapeDtypeStruct((_B, _MC, _KT), jnp.int32),
    )(xyzp, xyzpT)


def _sc_gather(tab, ids):
    mesh = plsc.VectorSubcoreMesh(core_axis_name="c", subcore_axis_name="s")

    nit = _PW // _CH

    @functools.partial(
        pl.kernel,
        mesh=mesh,
        out_type=jax.ShapeDtypeStruct((_N, _G), jnp.float32),
        scratch_types=[
            pltpu.VMEM((_PW,), jnp.int32),
            pltpu.VMEM((_CH, _G), jnp.float32),
            pltpu.VMEM((_CH, _G), jnp.float32),
            pltpu.SemaphoreType.DMA,
            pltpu.SemaphoreType.DMA,
            pltpu.SemaphoreType.DMA,
            pltpu.SemaphoreType.DMA,
        ],
    )
    def gathered(tab_hbm, ids_hbm, out_hbm, idx_all, buf0, buf1,
                 gs0, gs1, ss0, ss1):
        wid = lax.axis_index("s") * _NC + lax.axis_index("c")
        base = wid * _PW
        pltpu.sync_copy(ids_hbm.at[pl.ds(base, _PW)], idx_all)
        bufs, gsems, ssems = (buf0, buf1), (gs0, gs1), (ss0, ss1)
        gh = [None] * nit
        sh = [None] * nit
        for i in range(nit):
            b = i & 1
            if i >= 2:
                sh[i - 2].wait()
            gh[i] = pltpu.async_copy(
                tab_hbm.at[idx_all.at[pl.ds(i * _CH, _CH)]], bufs[b], gsems[b])
            if i >= 1:
                gh[i - 1].wait()
                sh[i - 1] = pltpu.async_copy(
                    bufs[1 - b],
                    out_hbm.at[pl.ds(base + (i - 1) * _CH, _CH)],
                    ssems[1 - b])
        gh[nit - 1].wait()
        sh[nit - 1] = pltpu.async_copy(
            bufs[(nit - 1) & 1],
            out_hbm.at[pl.ds(base + (nit - 1) * _CH, _CH)],
            ssems[(nit - 1) & 1])
        sh[nit - 2].wait()
        sh[nit - 1].wait()

    return gathered(tab, ids)


def _attn_body(q_ref, resi_ref, g_ref, p1c_ref,
               pb1_ref, pw2_ref, pb2_ref,
               w1c_ref, b1c_ref, aw2_ref, b2c_ref,
               ow_ref, ob_ref, rw_ref, rb_ref,
               *out_refs):
    g = g_ref[0, 0]
    kl = g[:, 0:_DT]
    vl = g[:, _DT:2 * _DT]
    p1g = g[:, 2 * _DT:3 * _DT]
    p1c = p1c_ref[0]
    p1cb = jnp.broadcast_to(p1c[:, None, :], (_T, _KT, _DT)).reshape(_R, _DT)
    p1 = jax.nn.relu(p1g - p1cb + pb1_ref[...])
    pos = _dot(p1, pw2_ref[...], 1, 1) + pb2_ref[...]
    q = q_ref[0]
    qb = jnp.broadcast_to(q[:, None, :], (_T, _KT, _DT)).reshape(_R, _DT)
    a0 = (qb - kl) + pos
    vp = vl + pos

    a0h = a0.astype(jnp.bfloat16)
    h = jax.nn.relu(_dot(a0h, w1c_ref[...], 1, 0) + b1c_ref[...])
    hh = h.astype(jnp.bfloat16)
    s = jnp.concatenate(
        [_dot(hh[:, i * 4 * _DT:(i + 1) * 4 * _DT], aw2_ref[i], 1, 1)
         for i in range(_UP)], axis=1) + b2c_ref[...]
    z = (s * 0.125).reshape(_T, _KT, _UP * _DT)
    zmax = jnp.max(z, axis=1, keepdims=True)
    e = jnp.exp(z - zmax)
    rden = 1.0 / jnp.sum(e, axis=1, keepdims=True)
    vp4 = jnp.concatenate([vp.reshape(_T, _KT, _DT)] * _UP, axis=2)
    f = jnp.sum(e * vp4, axis=1) * rden.reshape(_T, _UP * _DT)
    for i in range(_UP):
        fi = f[:, i * _DT:(i + 1) * _DT]
        o = _dot(ow_ref[i], fi, 1, 1) + ob_ref[i]
        r = _dot(rw_ref[i], resi_ref[0], 1, 1) + rb_ref[i]
        out_refs[i][0] = o + r


def _attention(qT, resiT, g4, p1T,
               pos_b1, pos_w2, pos_b2,
               attn_w1, attn_b1, attn_w2, attn_b2,
               out_w, out_b, res_w, res_b, c):
    grid = (_B, _MC // _T)
    toff = c * (_MC // _T)
    bt = lambda b, t: (b, toff + t, 0)
    bt4 = lambda b, t: (b, t, 0, 0)
    w = lambda *s: pl.BlockSpec(s, lambda b, t: (0,) * len(s))
    w1c = jnp.transpose(attn_w1, (2, 0, 1)).reshape(
        _DT, _UP * 4 * _DT).astype(jnp.bfloat16)
    b1c = attn_b1.reshape(1, _UP * 4 * _DT)
    b2c = attn_b2.reshape(1, _UP * _DT)
    w2h = attn_w2.astype(jnp.bfloat16)
    outs = pl.pallas_call(
        _attn_body,
        grid=grid,
        compiler_params=pltpu.CompilerParams(
            dimension_semantics=("parallel", "parallel")),
        in_specs=[
            pl.BlockSpec((1, _T, _DT), bt),
            pl.BlockSpec((1, _T, _D), bt),
            pl.BlockSpec((1, 1, _R, _G), bt4),
            pl.BlockSpec((1, _T, _DT), bt),
            w(1, _DT), w(_DT, _DT), w(1, _DT),
            w(_DT, _UP * 4 * _DT),
            w(1, _UP * 4 * _DT),
            w(_UP, _DT, 4 * _DT),
            w(1, _UP * _DT),
            w(_UP, _D, _DT),
            w(_UP, _D, 1),
            w(_UP, _D, _D),
            w(_UP, _D, 1),
        ],
        out_specs=[pl.BlockSpec((1, _D, _T), lambda b, t: (b, 0, t))
                   for _ in range(_UP)],
        out_shape=[jax.ShapeDtypeStruct((_B, _D, _MC), jnp.float32)
                   for _ in range(_UP)],
    )(qT, resiT, g4, p1T,
      pos_b1[None, :], pos_w2, pos_b2[None, :],
      w1c, b1c, w2h, b2c,
      out_w, out_b[:, :, None], res_w, res_b[:, :, None])
    return outs


def kernel(fts_q, fts_k, xyz, mv_w1, mv_b1, mv_w2, mv_b2, mv_wres, mv_bres,
           q_w, q_b, k_w, k_b, v_w, v_b, pos_w1, pos_b1, pos_w2, pos_b2,
           attn_w1, attn_b1, attn_w2, attn_b2, out_w, out_b, res_w, res_b):
    xyzp = jnp.pad(xyz, ((0, 0), (0, 0), (0, _XP - 3)))
    xyzpT = jnp.transpose(xyzp, (0, 2, 1))
    pos_w1p = jnp.pad(pos_w1, ((0, 0), (0, _XP - 3)))

    qT, tab, resiT = _preamble(fts_q, fts_k, xyzp, mv_w1, mv_b1, mv_w2,
                               mv_b2, mv_wres, mv_bres, q_w, q_b, k_w, k_b,
                               v_w, v_b, pos_w1p)
    p1T = tab[:, :, 2 * _DT:3 * _DT]
    tab_flat = tab.reshape(_B * _M, _G)

    chunk_outs = []
    for c in range(_PC):
        fids = _knn(xyzp, xyzpT, c)
        glg = _sc_gather(tab_flat, fids.reshape(_N))
        g4 = glg.reshape(_B, _MC // _T, _R, _G)
        chunk_outs.append(
            _attention(qT, resiT, g4, p1T,
                       pos_b1, pos_w2, pos_b2,
                       attn_w1, attn_b1, attn_w2, attn_b2,
                       out_w, out_b, res_w, res_b, c))

    return jnp.concatenate(
        [chunk_outs[c][h] for h in range(_UP) for c in range(_PC)], axis=2)


# speedup vs baseline: 1.0181x; 1.0181x over previous
import functools

import jax
import jax.numpy as jnp
from jax import lax
from jax.experimental import pallas as pl
from jax.experimental.pallas import tpu as pltpu
from jax.experimental.pallas import tpu_sc as plsc

_D = 128
_DT = 64
_KT = 16
_UP = 4
_B = 2
_M = 2048
_XP = 16
_G = 256
_TP = 512
_TK = 256
_T = 256
_R = _T * _KT

_NC = 2
_NS = 16
_NW = _NC * _NS
_PC = 2
_MC = _M // _PC
_N = _B * _MC * _KT
_PW = _N // _NW
_CH = 128


def _dot(a, b, ca, cb):
    return lax.dot_general(a, b, (((ca,), (cb,)), ((), ())),
                           preferred_element_type=jnp.float32)


def _preamble_body(fq_ref, fk_ref, xyz_ref, w1q_ref, w1k_ref, b1_ref, w2_ref,
                   b2_ref, wrq_ref, wrk_ref, br_ref, qw_ref, qb_ref, kw_ref,
                   kb_ref, vw_ref, vb_ref, pw1_ref,
                   q_out, tab_out, resi_out):
    fq = fq_ref[0]
    fk = fk_ref[0]
    h = _dot(w1q_ref[...], fq, 1, 0) + _dot(w1k_ref[...], fk, 1, 0)
    h = jax.nn.relu(h + b1_ref[...])
    fv = (_dot(h, w2_ref[...], 0, 1) + _dot(fq, wrq_ref[...], 0, 1)
          + _dot(fk, wrk_ref[...], 0, 1) + b2_ref[...] + br_ref[...])
    resi_out[0] = fv
    q_out[0] = _dot(fq, qw_ref[...], 0, 1) + qb_ref[...]
    kk = _dot(fk, kw_ref[...], 0, 1) + kb_ref[...]
    vv = _dot(fv, vw_ref[...], 1, 1) + vb_ref[...]
    p1 = _dot(xyz_ref[0], pw1_ref[...], 1, 1)
    tab_out[0] = jnp.concatenate(
        [kk, vv, p1, jnp.zeros((_TP, _DT), jnp.float32)], axis=1)


def _preamble(fts_q, fts_k, xyzp, mv_w1, mv_b1, mv_w2, mv_b2, mv_wres,
              mv_bres, q_w, q_b, k_w, k_b, v_w, v_b, pos_w1p):
    grid = (_B, _M // _TP)
    io = lambda b, t: (b, 0, t)
    w = lambda *s: pl.BlockSpec(s, lambda b, t: (0,) * len(s))
    return pl.pallas_call(
        _preamble_body,
        grid=grid,
        compiler_params=pltpu.CompilerParams(
            dimension_semantics=("parallel", "parallel")),
        in_specs=[
            pl.BlockSpec((1, _D, _TP), io),
            pl.BlockSpec((1, _D, _TP), io),
            pl.BlockSpec((1, _TP, _XP), lambda b, t: (b, t, 0)),
            w(_D, _D), w(_D, _D), w(_D, 1), w(_D, _D), w(1, _D),
            w(_D, _D), w(_D, _D), w(1, _D),
            w(_DT, _D), w(1, _DT), w(_DT, _D), w(1, _DT), w(_DT, _D), w(1, _DT),
            w(_DT, _XP),
        ],
        out_specs=[
            pl.BlockSpec((1, _TP, _DT), lambda b, t: (b, t, 0)),
            pl.BlockSpec((1, _TP, _G), lambda b, t: (b, t, 0)),
            pl.BlockSpec((1, _TP, _D), lambda b, t: (b, t, 0)),
        ],
        out_shape=[
            jax.ShapeDtypeStruct((_B, _M, _DT), jnp.float32),
            jax.ShapeDtypeStruct((_B, _M, _G), jnp.float32),
            jax.ShapeDtypeStruct((_B, _M, _D), jnp.float32),
        ],
    )(fts_q, fts_k, xyzp,
      mv_w1[:, :_D], mv_w1[:, _D:], mv_b1[:, None], mv_w2, mv_b2[None, :],
      mv_wres[:, :_D], mv_wres[:, _D:], mv_bres[None, :],
      q_w, q_b[None, :], k_w, k_b[None, :], v_w, v_b[None, :], pos_w1p)


def _knn_body(xt_ref, xa_ref, ids_out):
    b = pl.program_id(0)
    xt = xt_ref[0]
    xa = xa_ref[0]
    sqt = jnp.sum(xt * xt, axis=1, keepdims=True)
    sqa = jnp.sum(xa * xa, axis=0, keepdims=True)
    d = sqt + sqa - 2.0 * _dot(xt, xa, 1, 0)
    iota = lax.broadcasted_iota(jnp.int32, (_TK, _M), 1)
    cols = []
    for _ in range(_KT):
        idx = jnp.argmin(d, axis=1).astype(jnp.int32)[:, None]
        cols.append(idx)
        d = jnp.where(iota == idx, 3.0e38, d)
    ids = jnp.concatenate(cols, axis=1)
    ids_out[0] = ids + b * _M


def _knn(xyzp, xyzpT, c):
    toff = c * (_MC // _TK)
    return pl.pallas_call(
        _knn_body,
        grid=(_B, _MC // _TK),
        compiler_params=pltpu.CompilerParams(
            dimension_semantics=("parallel", "parallel")),
        in_specs=[
            pl.BlockSpec((1, _TK, _XP), lambda b, t: (b, toff + t, 0)),
            pl.BlockSpec((1, _XP, _M), lambda b, t: (b, 0, 0)),
        ],
        out_specs=pl.BlockSpec((1, _TK, _KT), lambda b, t: (b, t, 0)),
        out_shape=jax.ShapeDtypeStruct((_B, _MC, _KT), jnp.int32),
    )(xyzp, xyzpT)


def _sc_gather(tab, ids):
    mesh = plsc.VectorSubcoreMesh(core_axis_name="c", subcore_axis_name="s")

    @functools.partial(
        pl.kernel,
        mesh=mesh,
        out_type=jax.ShapeDtypeStruct((_N, _G), jnp.float32),
        scratch_types=[
            pltpu.VMEM((_CH,), jnp.int32),
            pltpu.VMEM((_CH, _G), jnp.float32),
            pltpu.SemaphoreType.DMA,
        ],
    )
    def gathered(tab_hbm, ids_hbm, out_hbm, idx_v, buf, sem):
        wid = lax.axis_index("s") * _NC + lax.axis_index("c")
        base = wid * _PW

        def chunk(c, carry):
            off = base + c * _CH
            pltpu.sync_copy(ids_hbm.at[pl.ds(off, _CH)], idx_v)
            pltpu.async_copy(tab_hbm.at[idx_v], buf, sem).wait()
            pltpu.sync_copy(buf, out_hbm.at[pl.ds(off, _CH)])
            return carry

        lax.fori_loop(0, _PW // _CH, chunk, 0)

    return gathered(tab, ids)


def _attn_body(q_ref, resi_ref, g_ref, p1c_ref,
               pb1_ref, pw2_ref, pb2_ref,
               w1c_ref, b1c_ref, aw2_ref, b2c_ref,
               ow_ref, ob_ref, rw_ref, rb_ref,
               *out_refs):
    g = g_ref[0, 0]
    kl = g[:, 0:_DT]
    vl = g[:, _DT:2 * _DT]
    p1g = g[:, 2 * _DT:3 * _DT]
    p1c = p1c_ref[0]
    p1cb = jnp.broadcast_to(p1c[:, None, :], (_T, _KT, _DT)).reshape(_R, _DT)
    p1 = jax.nn.relu(p1g - p1cb + pb1_ref[...])
    pos = _dot(p1, pw2_ref[...], 1, 1) + pb2_ref[...]
    q = q_ref[0]
    qb = jnp.broadcast_to(q[:, None, :], (_T, _KT, _DT)).reshape(_R, _DT)
    a0 = (qb - kl) + pos
    vp = vl + pos

    a0h = a0.astype(jnp.bfloat16)
    h = jax.nn.relu(_dot(a0h, w1c_ref[...], 1, 0) + b1c_ref[...])
    hh = h.astype(jnp.bfloat16)
    s = jnp.concatenate(
        [_dot(hh[:, i * 4 * _DT:(i + 1) * 4 * _DT], aw2_ref[i], 1, 1)
         for i in range(_UP)], axis=1) + b2c_ref[...]
    z = (s * 0.125).reshape(_T, _KT, _UP * _DT)
    zmax = jnp.max(z, axis=1, keepdims=True)
    e = jnp.exp(z - zmax)
    rden = 1.0 / jnp.sum(e, axis=1, keepdims=True)
    vp4 = jnp.concatenate([vp.reshape(_T, _KT, _DT)] * _UP, axis=2)
    f = jnp.sum(e * vp4, axis=1) * rden.reshape(_T, _UP * _DT)
    for i in range(_UP):
        fi = f[:, i * _DT:(i + 1) * _DT]
        o = _dot(ow_ref[i], fi, 1, 1) + ob_ref[i]
        r = _dot(rw_ref[i], resi_ref[0], 1, 1) + rb_ref[i]
        out_refs[i][0] = o + r


def _attention(qT, resiT, g4, p1T,
               pos_b1, pos_w2, pos_b2,
               attn_w1, attn_b1, attn_w2, attn_b2,
               out_w, out_b, res_w, res_b, c):
    grid = (_B, _MC // _T)
    toff = c * (_MC // _T)
    bt = lambda b, t: (b, toff + t, 0)
    bt4 = lambda b, t: (b, t, 0, 0)
    w = lambda *s: pl.BlockSpec(s, lambda b, t: (0,) * len(s))
    w1c = jnp.transpose(attn_w1, (2, 0, 1)).reshape(
        _DT, _UP * 4 * _DT).astype(jnp.bfloat16)
    b1c = attn_b1.reshape(1, _UP * 4 * _DT)
    b2c = attn_b2.reshape(1, _UP * _DT)
    w2h = attn_w2.astype(jnp.bfloat16)
    outs = pl.pallas_call(
        _attn_body,
        grid=grid,
        compiler_params=pltpu.CompilerParams(
            dimension_semantics=("parallel", "parallel")),
        in_specs=[
            pl.BlockSpec((1, _T, _DT), bt),
            pl.BlockSpec((1, _T, _D), bt),
            pl.BlockSpec((1, 1, _R, _G), bt4),
            pl.BlockSpec((1, _T, _DT), bt),
            w(1, _DT), w(_DT, _DT), w(1, _DT),
            w(_DT, _UP * 4 * _DT),
            w(1, _UP * 4 * _DT),
            w(_UP, _DT, 4 * _DT),
            w(1, _UP * _DT),
            w(_UP, _D, _DT),
            w(_UP, _D, 1),
            w(_UP, _D, _D),
            w(_UP, _D, 1),
        ],
        out_specs=[pl.BlockSpec((1, _D, _T), lambda b, t: (b, 0, t))
                   for _ in range(_UP)],
        out_shape=[jax.ShapeDtypeStruct((_B, _D, _MC), jnp.float32)
                   for _ in range(_UP)],
    )(qT, resiT, g4, p1T,
      pos_b1[None, :], pos_w2, pos_b2[None, :],
      w1c, b1c, w2h, b2c,
      out_w, out_b[:, :, None], res_w, res_b[:, :, None])
    return outs


def kernel(fts_q, fts_k, xyz, mv_w1, mv_b1, mv_w2, mv_b2, mv_wres, mv_bres,
           q_w, q_b, k_w, k_b, v_w, v_b, pos_w1, pos_b1, pos_w2, pos_b2,
           attn_w1, attn_b1, attn_w2, attn_b2, out_w, out_b, res_w, res_b):
    xyzp = jnp.pad(xyz, ((0, 0), (0, 0), (0, _XP - 3)))
    xyzpT = jnp.transpose(xyzp, (0, 2, 1))
    pos_w1p = jnp.pad(pos_w1, ((0, 0), (0, _XP - 3)))

    qT, tab, resiT = _preamble(fts_q, fts_k, xyzp, mv_w1, mv_b1, mv_w2,
                               mv_b2, mv_wres, mv_bres, q_w, q_b, k_w, k_b,
                               v_w, v_b, pos_w1p)
    p1T = tab[:, :, 2 * _DT:3 * _DT]
    tab_flat = tab.reshape(_B * _M, _G)

    chunk_outs = []
    for c in range(_PC):
        fids = _knn(xyzp, xyzpT, c)
        glg = _sc_gather(tab_flat, fids.reshape(_N))
        g4 = glg.reshape(_B, _MC // _T, _R, _G)
        chunk_outs.append(
            _attention(qT, resiT, g4, p1T,
                       pos_b1, pos_w2, pos_b2,
                       attn_w1, attn_b1, attn_w2, attn_b2,
                       out_w, out_b, res_w, res_b, c))

    return jnp.concatenate(
        [chunk_outs[c][h] for h in range(_UP) for c in range(_PC)], axis=2)
